# Initial kernel scaffold; baseline (speedup 1.0000x reference)
import functools

import jax
import jax.numpy as jnp
from jax import lax
from jax.experimental import pallas as pl
from jax.experimental.pallas import tpu as pltpu
from jax.experimental.pallas import tpu_sc as plsc

P = 7
H = 32
W = 32
C = 192
NLVL = 9
TROWS = NLVL * H * W
ZROW = TROWS
TPAD = TROWS + 16
RPB = 4 * P * P
RPB_PAD = 200


def _build_tables(fmap):
    def body(f_ref, out_ref):
        x = f_ref[...]
        xt = jnp.transpose(x.reshape(C, H * W))
        x3 = xt.reshape(H, W, C)

        def shift_h(a, d):
            tail = jnp.broadcast_to(a[H - 1 :], (d, W, C))
            return jnp.concatenate([a[d:], tail], axis=0)

        def shift_w(a, d):
            tail = jnp.broadcast_to(a[:, W - 1 :], (H, d, C))
            return jnp.concatenate([a[:, d:], tail], axis=1)

        h0 = x3
        h1 = jnp.maximum(h0, shift_h(h0, 1))
        h2 = jnp.maximum(h1, shift_h(h1, 2))
        for kh, hk in enumerate((h0, h1, h2)):
            w0 = hk
            w1 = jnp.maximum(w0, shift_w(w0, 1))
            w2 = jnp.maximum(w1, shift_w(w1, 2))
            for kw, wk in enumerate((w0, w1, w2)):
                lvl = kh * 3 + kw
                out_ref[pl.ds(lvl * H * W, H * W), :] = wk.reshape(H * W, C)
        out_ref[pl.ds(TROWS, TPAD - TROWS), :] = jnp.zeros(
            (TPAD - TROWS, C), jnp.float32
        )

    return pl.pallas_call(
        body,
        out_shape=jax.ShapeDtypeStruct((TPAD, C), jnp.float32),
    )(fmap)


def _corner_indices(boxes_p, scale):
    npad = boxes_p.shape[0]
    r = jnp.round(boxes_p * scale).astype(jnp.int32)
    rsw, rsh, rew, reh = r[:, 0], r[:, 1], r[:, 2], r[:, 3]
    roi_w = jnp.maximum(rew - rsw + 1, 1)
    roi_h = jnp.maximum(reh - rsh + 1, 1)
    b = jnp.arange(P, dtype=jnp.int32)

    def seg(rs, roi, lim):
        start = jnp.clip(b[None, :] * roi[:, None] // P + rs[:, None], 0, lim)
        end = jnp.clip(
            ((b[None, :] + 1) * roi[:, None] + (P - 1)) // P + rs[:, None], 0, lim
        )
        ln = end - start
        empty = ln <= 0
        k = (ln >= 2).astype(jnp.int32) + (ln >= 4).astype(jnp.int32)
        hi = end - (1 << k)
        return start, hi, k, empty

    ha, hb, kh, eh = seg(rsh, roi_h, H)
    wa, wb, kw, ew = seg(rsw, roi_w, W)

    base = (kh[:, :, None] * 3 + kw[:, None, :]) * (H * W)
    hai = ha[:, :, None] * W
    hbi = hb[:, :, None] * W
    wai = wa[:, None, :]
    wbi = wb[:, None, :]
    i0 = base + hai + wai
    i1 = base + hai + wbi
    i2 = base + hbi + wai
    i3 = base + hbi + wbi
    idx4 = jnp.stack([i0, i1, i2, i3], axis=-1)
    empty = eh[:, :, None] | ew[:, None, :]
    idx4 = jnp.where(empty[..., None], ZROW, idx4)
    idx4 = jnp.clip(idx4, 0, ZROW)
    out = jnp.full((npad, RPB_PAD), ZROW, jnp.int32)
    return out.at[:, :RPB].set(idx4.reshape(npad, RPB))


def _sc_gather_max(table, idx, npad, boxes_per_tile):
    mesh = plsc.VectorSubcoreMesh(core_axis_name="c", subcore_axis_name="s")

    @functools.partial(
        pl.kernel,
        mesh=mesh,
        out_type=jax.ShapeDtypeStruct((npad, P * P, C), jnp.float32),
        scratch_types=[
            pltpu.VMEM((RPB_PAD,), jnp.int32),
            pltpu.VMEM((RPB_PAD, C), jnp.float32),
            pltpu.VMEM((P * P, C), jnp.float32),
            pltpu.SemaphoreType.DMA,
        ],
    )
    def k(table_hbm, idx_hbm, out_hbm, idx_v, gbuf, obuf, sem):
        wid = lax.axis_index("s") * 2 + lax.axis_index("c")

        def box_body(bi, carry):
            box = wid * boxes_per_tile + bi
            pltpu.sync_copy(idx_hbm.at[box], idx_v)
            pltpu.async_copy(table_hbm.at[idx_v], gbuf, sem).wait()

            def s_body(s, c2):
                r = s * 4
                for kk in range(C // 16):
                    sl = pl.ds(kk * 16, 16)
                    m = jnp.maximum(
                        jnp.maximum(gbuf[r, sl], gbuf[r + 1, sl]),
                        jnp.maximum(gbuf[r + 2, sl], gbuf[r + 3, sl]),
                    )
                    obuf[s, sl] = m
                return c2

            lax.fori_loop(0, P * P, s_body, 0)
            pltpu.sync_copy(obuf, out_hbm.at[box])
            return carry

        lax.fori_loop(0, boxes_per_tile, box_body, 0)

    return k(table, idx)


def _transpose_out(gathered, n):
    blk = 8

    def body(in_ref, out_ref):
        out_ref[...] = jnp.transpose(in_ref[...], (0, 2, 1))

    return pl.pallas_call(
        body,
        grid=(n // blk,),
        in_specs=[pl.BlockSpec((blk, P * P, C), lambda i: (i, 0, 0))],
        out_specs=pl.BlockSpec((blk, C, P * P), lambda i: (i, 0, 0)),
        out_shape=jax.ShapeDtypeStruct((n, C, P * P), jnp.float32),
    )(gathered)


def kernel(feature, boxes, image_size):
    n = boxes.shape[0]
    ih = image_size[0].astype(jnp.float32)
    iw = image_size[1].astype(jnp.float32)
    scale = jnp.minimum(jnp.float32(H), jnp.float32(W)) / jnp.minimum(ih, iw)

    nw = 32
    boxes_per_tile = -(-n // nw)
    npad = nw * boxes_per_tile
    boxes_p = jnp.zeros((npad, 4), jnp.float32).at[:n].set(boxes)

    table = _build_tables(feature[0])
    idx = _corner_indices(boxes_p, scale)
    gathered = _sc_gather_max(table, idx, npad, boxes_per_tile)
    out = _transpose_out(gathered[:n], n)
    return out.reshape(n, C, P, P)


# speedup vs baseline: 4.6546x; 4.6546x over previous
import functools

import jax
import jax.numpy as jnp
from jax import lax
from jax.experimental import pallas as pl
from jax.experimental.pallas import tpu as pltpu
from jax.experimental.pallas import tpu_sc as plsc

P = 7
H = 32
W = 32
C = 192
CPAD = 256
NLVL = 9
TROWS = NLVL * H * W
ZROW = TROWS
TPAD = TROWS + 16
RPB = 4 * P * P
RPB_PAD = 208
GCH = 104


def _build_tables(fmap):
    def body(f_ref, out_ref):
        x = f_ref[...]
        xt = jnp.transpose(x.reshape(C, H * W))
        x3 = xt.reshape(H, W, C)

        def shift_h(a, d):
            tail = jnp.broadcast_to(a[H - 1 :], (d, W, C))
            return jnp.concatenate([a[d:], tail], axis=0)

        def shift_w(a, d):
            tail = jnp.broadcast_to(a[:, W - 1 :], (H, d, C))
            return jnp.concatenate([a[:, d:], tail], axis=1)

        h0 = x3
        h1 = jnp.maximum(h0, shift_h(h0, 1))
        h2 = jnp.maximum(h1, shift_h(h1, 2))
        for kh, hk in enumerate((h0, h1, h2)):
            w0 = hk
            w1 = jnp.maximum(w0, shift_w(w0, 1))
            w2 = jnp.maximum(w1, shift_w(w1, 2))
            for kw, wk in enumerate((w0, w1, w2)):
                lvl = kh * 3 + kw
                wkp = jnp.concatenate(
                    [wk.reshape(H * W, C), jnp.zeros((H * W, CPAD - C), jnp.float32)],
                    axis=1,
                )
                out_ref[pl.ds(lvl * H * W, H * W), :] = wkp
        out_ref[pl.ds(TROWS, TPAD - TROWS), :] = jnp.zeros(
            (TPAD - TROWS, CPAD), jnp.float32
        )

    return pl.pallas_call(
        body,
        out_shape=jax.ShapeDtypeStruct((TPAD, CPAD), jnp.float32),
    )(fmap)


def _corner_indices(boxes_p, scale):
    npad = boxes_p.shape[0]
    r = jnp.round(boxes_p * scale).astype(jnp.int32)
    rsw, rsh, rew, reh = r[:, 0], r[:, 1], r[:, 2], r[:, 3]
    roi_w = jnp.maximum(rew - rsw + 1, 1)
    roi_h = jnp.maximum(reh - rsh + 1, 1)
    b = jnp.arange(P, dtype=jnp.int32)

    def seg(rs, roi, lim):
        start = jnp.clip(b[None, :] * roi[:, None] // P + rs[:, None], 0, lim)
        end = jnp.clip(
            ((b[None, :] + 1) * roi[:, None] + (P - 1)) // P + rs[:, None], 0, lim
        )
        ln = end - start
        empty = ln <= 0
        k = (ln >= 2).astype(jnp.int32) + (ln >= 4).astype(jnp.int32)
        hi = end - (1 << k)
        return start, hi, k, empty

    ha, hb, kh, eh = seg(rsh, roi_h, H)
    wa, wb, kw, ew = seg(rsw, roi_w, W)

    base = (kh[:, :, None] * 3 + kw[:, None, :]) * (H * W)
    hai = ha[:, :, None] * W
    hbi = hb[:, :, None] * W
    wai = wa[:, None, :]
    wbi = wb[:, None, :]
    i0 = base + hai + wai
    i1 = base + hai + wbi
    i2 = base + hbi + wai
    i3 = base + hbi + wbi
    idx4 = jnp.stack([i0, i1, i2, i3], axis=-1)
    empty = eh[:, :, None] | ew[:, None, :]
    idx4 = jnp.where(empty[..., None], ZROW, idx4)
    idx4 = jnp.clip(idx4, 0, ZROW)
    out = jnp.full((npad, RPB_PAD), ZROW, jnp.int32)
    out = out.at[:, :RPB].set(idx4.reshape(npad, RPB))
    return out.reshape(npad, 2, GCH)


def _sc_gather_max(table, idx, npad, boxes_per_tile):
    mesh = plsc.VectorSubcoreMesh(core_axis_name="c", subcore_axis_name="s")

    @functools.partial(
        pl.kernel,
        mesh=mesh,
        out_type=jax.ShapeDtypeStruct((npad, P * P, C), jnp.float32),
        scratch_types=[
            pltpu.VMEM((2, GCH), jnp.int32),
            pltpu.VMEM((GCH, CPAD), jnp.float32),
            pltpu.VMEM((GCH, CPAD), jnp.float32),
            pltpu.VMEM((P * P, C), jnp.float32),
            pltpu.SemaphoreType.DMA,
        ],
    )
    def k(table_hbm, idx_hbm, out_hbm, idx_v, gbuf0, gbuf1, obuf, sem):
        wid = lax.axis_index("s") * 2 + lax.axis_index("c")

        def box_body(bi, carry):
            box = wid * boxes_per_tile + bi
            pltpu.sync_copy(idx_hbm.at[box], idx_v)
            pltpu.async_copy(table_hbm.at[idx_v.at[0]], gbuf0, sem).wait()
            pltpu.async_copy(table_hbm.at[idx_v.at[1]], gbuf1, sem).wait()

            def make_s_body(gbuf, s_off):
                def s_body(s, c2):
                    r = s * 4 - s_off * 4
                    for kk in range(C // 16):
                        sl = pl.ds(kk * 16, 16)
                        m = jnp.maximum(
                            jnp.maximum(gbuf[r, sl], gbuf[r + 1, sl]),
                            jnp.maximum(gbuf[r + 2, sl], gbuf[r + 3, sl]),
                        )
                        obuf[s, sl] = m
                    return c2

                return s_body

            lax.fori_loop(0, GCH // 4, make_s_body(gbuf0, 0), 0)
            lax.fori_loop(GCH // 4, P * P, make_s_body(gbuf1, GCH // 4), 0)
            pltpu.sync_copy(obuf, out_hbm.at[box])
            return carry

        lax.fori_loop(0, boxes_per_tile, box_body, 0)

    return k(table, idx)


def _transpose_out(gathered, n):
    blk = 8

    def body(in_ref, out_ref):
        out_ref[...] = jnp.transpose(in_ref[...], (0, 2, 1))

    return pl.pallas_call(
        body,
        grid=(n // blk,),
        in_specs=[pl.BlockSpec((blk, P * P, C), lambda i: (i, 0, 0))],
        out_specs=pl.BlockSpec((blk, C, P * P), lambda i: (i, 0, 0)),
        out_shape=jax.ShapeDtypeStruct((n, C, P * P), jnp.float32),
    )(gathered)


def kernel(feature, boxes, image_size):
    n = boxes.shape[0]
    ih = image_size[0].astype(jnp.float32)
    iw = image_size[1].astype(jnp.float32)
    scale = jnp.minimum(jnp.float32(H), jnp.float32(W)) / jnp.minimum(ih, iw)

    nw = 32
    boxes_per_tile = -(-n // nw)
    npad = nw * boxes_per_tile
    boxes_p = jnp.zeros((npad, 4), jnp.float32).at[:n].set(boxes)

    table = _build_tables(feature[0])
    idx = _corner_indices(boxes_p, scale)
    gathered = _sc_gather_max(table, idx, npad, boxes_per_tile)
    out = _transpose_out(gathered[:n], n)
    return out.reshape(n, C, P, P)


# speedup vs baseline: 5.3880x; 1.1576x over previous
import functools

import jax
import jax.numpy as jnp
from jax import lax
from jax.experimental import pallas as pl
from jax.experimental.pallas import tpu as pltpu
from jax.experimental.pallas import tpu_sc as plsc

P = 7
H = 32
W = 32
C = 192
CH = 96
CW = 128
NLVL = 9
TROWS = NLVL * H * W
ZROW = TROWS
TPAD = TROWS + 16
RPB = 4 * P * P
RPB_PAD = 208
GCH = 104
NW = 32
BPT = 32
DEPTH = 4


def _pack(wk):
    lo = lax.bitcast_convert_type(wk[:, :CH].astype(jnp.bfloat16), jnp.uint16)
    hi = lax.bitcast_convert_type(wk[:, CH:C].astype(jnp.bfloat16), jnp.uint16)
    w = (hi.astype(jnp.uint32) << 16) | lo.astype(jnp.uint32)
    w = lax.bitcast_convert_type(w, jnp.float32)
    return jnp.concatenate(
        [w, jnp.zeros((wk.shape[0], CW - CH), jnp.float32)], axis=1
    )


def _build_tables(fmap):
    def body(f_ref, out_ref):
        x = f_ref[...]
        xt = jnp.transpose(x.reshape(C, H * W))
        x3 = xt.reshape(H, W, C)

        def shift_h(a, d):
            tail = jnp.broadcast_to(a[H - 1 :], (d, W, C))
            return jnp.concatenate([a[d:], tail], axis=0)

        def shift_w(a, d):
            tail = jnp.broadcast_to(a[:, W - 1 :], (H, d, C))
            return jnp.concatenate([a[:, d:], tail], axis=1)

        h0 = x3
        h1 = jnp.maximum(h0, shift_h(h0, 1))
        h2 = jnp.maximum(h1, shift_h(h1, 2))
        for kh, hk in enumerate((h0, h1, h2)):
            w0 = hk
            w1 = jnp.maximum(w0, shift_w(w0, 1))
            w2 = jnp.maximum(w1, shift_w(w1, 2))
            for kw, wk in enumerate((w0, w1, w2)):
                lvl = kh * 3 + kw
                out_ref[pl.ds(lvl * H * W, H * W), :] = _pack(wk.reshape(H * W, C))
        out_ref[pl.ds(TROWS, TPAD - TROWS), :] = jnp.zeros(
            (TPAD - TROWS, CW), jnp.float32
        )

    return pl.pallas_call(
        body,
        out_shape=jax.ShapeDtypeStruct((TPAD, CW), jnp.float32),
    )(fmap)


def _corner_indices(boxes_p, scale):
    npad = boxes_p.shape[0]
    r = jnp.round(boxes_p * scale).astype(jnp.int32)
    rsw, rsh, rew, reh = r[:, 0], r[:, 1], r[:, 2], r[:, 3]
    roi_w = jnp.maximum(rew - rsw + 1, 1)
    roi_h = jnp.maximum(reh - rsh + 1, 1)
    b = jnp.arange(P, dtype=jnp.int32)

    def seg(rs, roi, lim):
        start = jnp.clip(b[None, :] * roi[:, None] // P + rs[:, None], 0, lim)
        end = jnp.clip(
            ((b[None, :] + 1) * roi[:, None] + (P - 1)) // P + rs[:, None], 0, lim
        )
        ln = end - start
        empty = ln <= 0
        k = (ln >= 2).astype(jnp.int32) + (ln >= 4).astype(jnp.int32)
        hi = end - (1 << k)
        return start, hi, k, empty

    ha, hb, kh, eh = seg(rsh, roi_h, H)
    wa, wb, kw, ew = seg(rsw, roi_w, W)

    base = (kh[:, :, None] * 3 + kw[:, None, :]) * (H * W)
    hai = ha[:, :, None] * W
    hbi = hb[:, :, None] * W
    wai = wa[:, None, :]
    wbi = wb[:, None, :]
    i0 = base + hai + wai
    i1 = base + hai + wbi
    i2 = base + hbi + wai
    i3 = base + hbi + wbi
    idx4 = jnp.stack([i0, i1, i2, i3], axis=-1)
    empty = eh[:, :, None] | ew[:, None, :]
    idx4 = jnp.where(empty[..., None], ZROW, idx4)
    idx4 = jnp.clip(idx4, 0, ZROW)
    out = jnp.full((npad, RPB_PAD), ZROW, jnp.int32)
    out = out.at[:, :RPB].set(idx4.reshape(npad, RPB))
    return out.reshape(npad, 2, GCH)


def _sc_gather_max(table, idx, npad):
    mesh = plsc.VectorSubcoreMesh(core_axis_name="c", subcore_axis_name="s")

    @functools.partial(
        pl.kernel,
        mesh=mesh,
        compiler_params=pltpu.CompilerParams(needs_layout_passes=False),
        out_type=jax.ShapeDtypeStruct((npad, P * P, CW), jnp.float32),
        scratch_types=[
            pltpu.VMEM((BPT, 2, GCH), jnp.int32),
            pltpu.VMEM((GCH, CW), jnp.float32),
            pltpu.VMEM((GCH, CW), jnp.float32),
            pltpu.VMEM((GCH, CW), jnp.float32),
            pltpu.VMEM((GCH, CW), jnp.float32),
            pltpu.VMEM((P * P, CW), jnp.float32),
            pltpu.VMEM((P * P, CW), jnp.float32),
            pltpu.SemaphoreType.DMA,
            pltpu.SemaphoreType.DMA,
            pltpu.SemaphoreType.DMA,
            pltpu.SemaphoreType.DMA,
            pltpu.SemaphoreType.DMA,
            pltpu.SemaphoreType.DMA,
        ],
    )
    def k(table_hbm, idx_hbm, out_hbm, idx_v,
          sl0, sl1, sl2, sl3, ob0, ob1, s0, s1, s2, s3, oa, ob):
        wid = lax.axis_index("s") * 2 + lax.axis_index("c")
        slots = (sl0, sl1, sl2, sl3)
        obufs = (ob0, ob1)
        gsems = (s0, s1, s2, s3)
        osems = (oa, ob)
        base_box = wid * BPT

        pltpu.sync_copy(idx_hbm.at[pl.ds(base_box, BPT)], idx_v)

        def issue(slot_i, box_local, half):
            pltpu.async_copy(
                table_hbm.at[idx_v.at[box_local, half]],
                slots[slot_i],
                gsems[slot_i],
            )

        for i in range(DEPTH):
            issue(i, i // 2, i % 2)

        def compute_chunk(slot_i, obuf_i, half):
            nbins = 26 if half == 0 else P * P - 26
            soff = 26 * half

            sbuf = slots[slot_i]
            obuf = obufs[obuf_i]

            def bin_body(sl, carry):
                rr = sl * 4
                for hh in range(CH // 16):
                    cs = pl.ds(hh * 16, 16)
                    g0 = plsc.bitcast(sbuf[rr, cs], jnp.bfloat16)
                    g1 = plsc.bitcast(sbuf[rr + 1, cs], jnp.bfloat16)
                    g2 = plsc.bitcast(sbuf[rr + 2, cs], jnp.bfloat16)
                    g3 = plsc.bitcast(sbuf[rr + 3, cs], jnp.bfloat16)
                    m = jnp.maximum(jnp.maximum(g0, g1), jnp.maximum(g2, g3))
                    obuf[soff + sl, cs] = plsc.bitcast(m, jnp.float32)
                return carry

            lax.fori_loop(0, nbins, bin_body, 0)

        nsteps = BPT // 2

        def step_body(g, carry):
            for i in range(DEPTH):
                obuf_i = i // 2
                pltpu.make_async_copy(
                    table_hbm.at[idx_v.at[0, 0]], slots[i], gsems[i]
                ).wait()
                if i % 2 == 0:
                    @pl.when(g > 0)
                    def _():
                        pltpu.make_async_copy(
                            obufs[obuf_i], out_hbm.at[0], osems[obuf_i]
                        ).wait()

                compute_chunk(i, obuf_i, i % 2)

                @pl.when(g < nsteps - 1)
                def _():
                    issue(i, (g + 1) * 2 + i // 2, i % 2)

                if i % 2 == 1:
                    box = base_box + g * 2 + obuf_i
                    pltpu.async_copy(
                        obufs[obuf_i], out_hbm.at[box], osems[obuf_i]
                    )
            return carry

        lax.fori_loop(0, nsteps, step_body, 0)
        for obuf_i in range(2):
            pltpu.make_async_copy(
                obufs[obuf_i], out_hbm.at[0], osems[obuf_i]
            ).wait()

    return k(table, idx)


def _transpose_out(gathered, n):
    blk = 8

    def body(in_ref, out_ref):
        xi = lax.bitcast_convert_type(in_ref[...][:, :, :CH], jnp.int32)
        lo = lax.bitcast_convert_type(xi << 16, jnp.float32)
        hi = lax.bitcast_convert_type(
            xi & jnp.int32(-65536), jnp.float32
        )
        x = jnp.concatenate([lo, hi], axis=2)
        out_ref[...] = jnp.transpose(x, (0, 2, 1))

    return pl.pallas_call(
        body,
        grid=(n // blk,),
        in_specs=[pl.BlockSpec((blk, P * P, CW), lambda i: (i, 0, 0))],
        out_specs=pl.BlockSpec((blk, C, P * P), lambda i: (i, 0, 0)),
        out_shape=jax.ShapeDtypeStruct((n, C, P * P), jnp.float32),
    )(gathered)


def kernel(feature, boxes, image_size):
    n = boxes.shape[0]
    ih = image_size[0].astype(jnp.float32)
    iw = image_size[1].astype(jnp.float32)
    scale = jnp.minimum(jnp.float32(H), jnp.float32(W)) / jnp.minimum(ih, iw)

    npad = NW * BPT
    boxes_p = jnp.zeros((npad, 4), jnp.float32).at[:n].set(boxes)

    table = _build_tables(feature[0])
    idx = _corner_indices(boxes_p, scale)
    gathered = _sc_gather_max(table, idx, npad)
    out = _transpose_out(gathered[:n], n)
    return out.reshape(n, C, P, P)


# speedup vs baseline: 11.0318x; 2.0475x over previous
import functools

import jax
import jax.numpy as jnp
from jax import lax
from jax.experimental import pallas as pl
from jax.experimental.pallas import tpu as pltpu
from jax.experimental.pallas import tpu_sc as plsc

P = 7
H = 32
W = 32
C = 192
CH = 96
CW = 128
NLVL = 9
TROWS = NLVL * H * W
NZROWS = 256
ZROW = TROWS
TPAD = TROWS + NZROWS
RPB = 4 * P * P
RPB_PAD = 208
GCH = 104
NW = 32
BPT = 32
DEPTH = 4


def _pack(wk):
    lo = lax.bitcast_convert_type(wk[:, :CH].astype(jnp.bfloat16), jnp.uint16)
    hi = lax.bitcast_convert_type(wk[:, CH:C].astype(jnp.bfloat16), jnp.uint16)
    w = (hi.astype(jnp.uint32) << 16) | lo.astype(jnp.uint32)
    w = lax.bitcast_convert_type(w, jnp.float32)
    return jnp.concatenate(
        [w, jnp.zeros((wk.shape[0], CW - CH), jnp.float32)], axis=1
    )


def _build_tables(fmap):
    def body(f_ref, out_ref):
        x = f_ref[...]
        xt = jnp.transpose(x.reshape(C, H * W))
        x3 = xt.reshape(H, W, C)

        def shift_h(a, d):
            tail = jnp.broadcast_to(a[H - 1 :], (d, W, C))
            return jnp.concatenate([a[d:], tail], axis=0)

        def shift_w(a, d):
            tail = jnp.broadcast_to(a[:, W - 1 :], (H, d, C))
            return jnp.concatenate([a[:, d:], tail], axis=1)

        h0 = x3
        h1 = jnp.maximum(h0, shift_h(h0, 1))
        h2 = jnp.maximum(h1, shift_h(h1, 2))
        for kh, hk in enumerate((h0, h1, h2)):
            w0 = hk
            w1 = jnp.maximum(w0, shift_w(w0, 1))
            w2 = jnp.maximum(w1, shift_w(w1, 2))
            for kw, wk in enumerate((w0, w1, w2)):
                lvl = kh * 3 + kw
                out_ref[pl.ds(lvl * H * W, H * W), :] = _pack(wk.reshape(H * W, C))
        out_ref[pl.ds(TROWS, NZROWS), :] = jnp.zeros((NZROWS, CW), jnp.float32)

    return pl.pallas_call(
        body,
        out_shape=jax.ShapeDtypeStruct((TPAD, CW), jnp.float32),
    )(fmap)


def _corner_indices(boxes_p, scale):
    npad = boxes_p.shape[0]
    r = jnp.round(boxes_p * scale).astype(jnp.int32)
    rsw, rsh, rew, reh = r[:, 0], r[:, 1], r[:, 2], r[:, 3]
    roi_w = jnp.maximum(rew - rsw + 1, 1)
    roi_h = jnp.maximum(reh - rsh + 1, 1)
    b = jnp.arange(P, dtype=jnp.int32)

    def seg(rs, roi, lim):
        start = jnp.clip(b[None, :] * roi[:, None] // P + rs[:, None], 0, lim)
        end = jnp.clip(
            ((b[None, :] + 1) * roi[:, None] + (P - 1)) // P + rs[:, None], 0, lim
        )
        ln = end - start
        empty = ln <= 0
        k = (ln >= 2).astype(jnp.int32) + (ln >= 4).astype(jnp.int32)
        hi = end - (1 << k)
        return start, hi, k, empty

    ha, hb, kh, eh = seg(rsh, roi_h, H)
    wa, wb, kw, ew = seg(rsw, roi_w, W)

    base = (kh[:, :, None] * 3 + kw[:, None, :]) * (H * W)
    hai = ha[:, :, None] * W
    hbi = hb[:, :, None] * W
    wai = wa[:, None, :]
    wbi = wb[:, None, :]
    i0 = base + hai + wai
    i1 = base + hai + wbi
    i2 = base + hbi + wai
    i3 = base + hbi + wbi
    idx4 = jnp.stack([i0, i1, i2, i3], axis=-1)
    empty = eh[:, :, None] | ew[:, None, :]
    slot = jnp.arange(RPB_PAD, dtype=jnp.int32)[None, :]
    boxn = jnp.arange(npad, dtype=jnp.int32)[:, None]
    zspread = ZROW + ((boxn * 83 + slot) & (NZROWS - 1))
    zs4 = zspread[:, :RPB].reshape(npad, P, P, 4)
    idx4 = jnp.where(empty[..., None], zs4, idx4)
    idx4 = jnp.clip(idx4, 0, TPAD - 1)
    out = zspread.at[:, :RPB].set(idx4.reshape(npad, RPB))
    return out.reshape(npad, 2, GCH)


def _sc_gather_max(table, idx, npad):
    mesh = plsc.VectorSubcoreMesh(core_axis_name="c", subcore_axis_name="s")

    @functools.partial(
        pl.kernel,
        mesh=mesh,
        compiler_params=pltpu.CompilerParams(needs_layout_passes=False),
        out_type=jax.ShapeDtypeStruct((npad, P * P, CW), jnp.float32),
        scratch_types=[
            pltpu.VMEM((BPT, 2, GCH), jnp.int32),
            pltpu.VMEM((GCH, CW), jnp.float32),
            pltpu.VMEM((GCH, CW), jnp.float32),
            pltpu.VMEM((GCH, CW), jnp.float32),
            pltpu.VMEM((GCH, CW), jnp.float32),
            pltpu.VMEM((P * P, CW), jnp.float32),
            pltpu.VMEM((P * P, CW), jnp.float32),
            pltpu.SemaphoreType.DMA,
            pltpu.SemaphoreType.DMA,
            pltpu.SemaphoreType.DMA,
            pltpu.SemaphoreType.DMA,
            pltpu.SemaphoreType.DMA,
            pltpu.SemaphoreType.DMA,
        ],
    )
    def k(table_hbm, idx_hbm, out_hbm, idx_v,
          sl0, sl1, sl2, sl3, ob0, ob1, s0, s1, s2, s3, oa, ob):
        wid = lax.axis_index("s") * 2 + lax.axis_index("c")
        slots = (sl0, sl1, sl2, sl3)
        obufs = (ob0, ob1)
        gsems = (s0, s1, s2, s3)
        osems = (oa, ob)
        base_box = wid * BPT

        pltpu.sync_copy(idx_hbm.at[pl.ds(base_box, BPT)], idx_v)

        def issue(slot_i, box_local, half):
            pltpu.async_copy(
                table_hbm.at[idx_v.at[box_local, half]],
                slots[slot_i],
                gsems[slot_i],
            )

        for i in range(DEPTH):
            issue(i, i // 2, i % 2)

        def compute_chunk(slot_i, obuf_i, half):
            nbins = 26 if half == 0 else P * P - 26
            soff = 26 * half

            sbuf = slots[slot_i]
            obuf = obufs[obuf_i]

            def bin_body(sl, carry):
                rr = sl * 4
                for hh in range(CH // 16):
                    cs = pl.ds(hh * 16, 16)
                    g0 = plsc.bitcast(sbuf[rr, cs], jnp.bfloat16)
                    g1 = plsc.bitcast(sbuf[rr + 1, cs], jnp.bfloat16)
                    g2 = plsc.bitcast(sbuf[rr + 2, cs], jnp.bfloat16)
                    g3 = plsc.bitcast(sbuf[rr + 3, cs], jnp.bfloat16)
                    m = jnp.maximum(jnp.maximum(g0, g1), jnp.maximum(g2, g3))
                    obuf[soff + sl, cs] = plsc.bitcast(m, jnp.float32)
                return carry

            lax.fori_loop(0, nbins, bin_body, 0)

        nsteps = BPT // 2

        def step_body(g, carry):
            for i in range(DEPTH):
                obuf_i = i // 2
                pltpu.make_async_copy(
                    table_hbm.at[idx_v.at[0, 0]], slots[i], gsems[i]
                ).wait()
                if i % 2 == 0:
                    @pl.when(g > 0)
                    def _():
                        pltpu.make_async_copy(
                            obufs[obuf_i], out_hbm.at[0], osems[obuf_i]
                        ).wait()

                compute_chunk(i, obuf_i, i % 2)

                @pl.when(g < nsteps - 1)
                def _():
                    issue(i, (g + 1) * 2 + i // 2, i % 2)

                if i % 2 == 1:
                    box = base_box + g * 2 + obuf_i
                    pltpu.async_copy(
                        obufs[obuf_i], out_hbm.at[box], osems[obuf_i]
                    )
            return carry

        lax.fori_loop(0, nsteps, step_body, 0)
        for obuf_i in range(2):
            pltpu.make_async_copy(
                obufs[obuf_i], out_hbm.at[0], osems[obuf_i]
            ).wait()

    return k(table, idx)


def _transpose_out(gathered, n):
    blk = 8

    def body(in_ref, out_ref):
        xi = lax.bitcast_convert_type(in_ref[...][:, :, :CH], jnp.int32)
        lo = lax.bitcast_convert_type(xi << 16, jnp.float32)
        hi = lax.bitcast_convert_type(
            xi & jnp.int32(-65536), jnp.float32
        )
        x = jnp.concatenate([lo, hi], axis=2)
        out_ref[...] = jnp.transpose(x, (0, 2, 1))

    return pl.pallas_call(
        body,
        grid=(n // blk,),
        in_specs=[pl.BlockSpec((blk, P * P, CW), lambda i: (i, 0, 0))],
        out_specs=pl.BlockSpec((blk, C, P * P), lambda i: (i, 0, 0)),
        out_shape=jax.ShapeDtypeStruct((n, C, P * P), jnp.float32),
    )(gathered)


def kernel(feature, boxes, image_size):
    n = boxes.shape[0]
    ih = image_size[0].astype(jnp.float32)
    iw = image_size[1].astype(jnp.float32)
    scale = jnp.minimum(jnp.float32(H), jnp.float32(W)) / jnp.minimum(ih, iw)

    npad = NW * BPT
    boxes_p = jnp.zeros((npad, 4), jnp.float32).at[:n].set(boxes)

    table = _build_tables(feature[0])
    idx = _corner_indices(boxes_p, scale)
    gathered = _sc_gather_max(table, idx, npad)
    out = _transpose_out(gathered[:n], n)
    return out.reshape(n, C, P, P)


# speedup vs baseline: 13.2389x; 1.2001x over previous
import functools

import jax
import jax.numpy as jnp
from jax import lax
from jax.experimental import pallas as pl
from jax.experimental.pallas import tpu as pltpu
from jax.experimental.pallas import tpu_sc as plsc

P = 7
H = 32
W = 32
C = 192
CH = 96
CW = 128
NLVL = 9
TROWS = NLVL * H * W
NZROWS = 256
ZROW = TROWS
TPAD = TROWS + NZROWS
RPB = 4 * P * P
RPB_PAD = 208
GCH = 104
NW = 32
BPT = 32
DEPTH = 4


def _idx_math(rb, boxn, slot, ph, pw, jj, n):
    rsw = rb[:, 0:1]
    rsh = rb[:, 1:2]
    rew = rb[:, 2:3]
    reh = rb[:, 3:4]
    roi_w = jnp.maximum(rew - rsw + 1, 1)
    roi_h = jnp.maximum(reh - rsh + 1, 1)
    inv7 = jnp.float32(1.0 / 7.0)

    def fdiv7(x):
        return jnp.floor((x.astype(jnp.float32) + 0.5) * inv7).astype(jnp.int32)

    def seg(pb, rs, roi, lim):
        start = jnp.clip(fdiv7(pb * roi) + rs, 0, lim)
        end = jnp.clip(fdiv7((pb + 1) * roi - 1) + 1 + rs, 0, lim)
        ln = end - start
        pw2 = jnp.where(ln >= 4, 4, jnp.where(ln >= 2, 2, 1))
        k = jnp.where(ln >= 4, 2, jnp.where(ln >= 2, 1, 0))
        return start, end - pw2, k, ln <= 0

    ha, hb, kh, eh = seg(ph, rsh, roi_h, H)
    wa, wb, kw, ew = seg(pw, rsw, roi_w, W)

    base = (kh * 3 + kw) * (H * W)
    ih = jnp.where(jj >= 2, hb, ha) * W
    iw = jnp.where((jj & 1) == 1, wb, wa)
    idx = base + ih + iw
    zspread = ZROW + ((boxn * 83 + slot) & (NZROWS - 1))
    bad = eh | ew | (boxn >= n) | (slot >= RPB)
    return jnp.clip(jnp.where(bad, zspread, idx), 0, TPAD - 1)


def _pack(wk):
    lo = lax.bitcast_convert_type(wk[:, :CH].astype(jnp.bfloat16), jnp.uint16)
    hi = lax.bitcast_convert_type(wk[:, CH:C].astype(jnp.bfloat16), jnp.uint16)
    w = (hi.astype(jnp.uint32) << 16) | lo.astype(jnp.uint32)
    w = lax.bitcast_convert_type(w, jnp.float32)
    return jnp.concatenate(
        [w, jnp.zeros((wk.shape[0], CW - CH), jnp.float32)], axis=1
    )


def _build_tables_and_idx(fmap, rbox, cols, n, npad):
    nslot = 256

    def body(f_ref, b_ref, c_ref, out_ref, idx_ref):
        rb = jnp.concatenate(
            [b_ref[...], jnp.zeros((npad - n, 4), jnp.int32)], axis=0
        )
        cc = c_ref[...]
        ph = jnp.broadcast_to(cc[0:1, :], (npad, nslot))
        pw = jnp.broadcast_to(cc[1:2, :], (npad, nslot))
        jj = jnp.broadcast_to(cc[2:3, :], (npad, nslot))
        boxn = lax.broadcasted_iota(jnp.int32, (npad, nslot), 0)
        slot = lax.broadcasted_iota(jnp.int32, (npad, nslot), 1)
        idx = _idx_math(rb, boxn, slot, ph, pw, jj, n)
        idx_ref[...] = idx[:, :RPB_PAD]

        _tables_body(f_ref, out_ref)

    return pl.pallas_call(
        body,
        out_shape=(
            jax.ShapeDtypeStruct((TPAD, CW), jnp.float32),
            jax.ShapeDtypeStruct((npad, RPB_PAD), jnp.int32),
        ),
    )(fmap, rbox, cols)


def _tables_body(f_ref, out_ref):
    x = f_ref[...]
    xt = jnp.transpose(x.reshape(C, H * W))
    x3 = xt.reshape(H, W, C)

    def shift_h(a, d):
        tail = jnp.broadcast_to(a[H - 1 :], (d, W, C))
        return jnp.concatenate([a[d:], tail], axis=0)

    def shift_w(a, d):
        tail = jnp.broadcast_to(a[:, W - 1 :], (H, d, C))
        return jnp.concatenate([a[:, d:], tail], axis=1)

    h0 = x3
    h1 = jnp.maximum(h0, shift_h(h0, 1))
    h2 = jnp.maximum(h1, shift_h(h1, 2))
    for kh, hk in enumerate((h0, h1, h2)):
        w0 = hk
        w1 = jnp.maximum(w0, shift_w(w0, 1))
        w2 = jnp.maximum(w1, shift_w(w1, 2))
        for kw, wk in enumerate((w0, w1, w2)):
            lvl = kh * 3 + kw
            out_ref[pl.ds(lvl * H * W, H * W), :] = _pack(wk.reshape(H * W, C))
    out_ref[pl.ds(TROWS, NZROWS), :] = jnp.zeros((NZROWS, CW), jnp.float32)


def _sc_gather_max(table, idx, npad):
    mesh = plsc.VectorSubcoreMesh(core_axis_name="c", subcore_axis_name="s")

    @functools.partial(
        pl.kernel,
        mesh=mesh,
        compiler_params=pltpu.CompilerParams(needs_layout_passes=False),
        out_type=jax.ShapeDtypeStruct((npad, P * P, CW), jnp.float32),
        scratch_types=[
            pltpu.VMEM((BPT, 2, GCH), jnp.int32),
            pltpu.VMEM((GCH, CW), jnp.float32),
            pltpu.VMEM((GCH, CW), jnp.float32),
            pltpu.VMEM((GCH, CW), jnp.float32),
            pltpu.VMEM((GCH, CW), jnp.float32),
            pltpu.VMEM((P * P, CW), jnp.float32),
            pltpu.VMEM((P * P, CW), jnp.float32),
            pltpu.SemaphoreType.DMA,
            pltpu.SemaphoreType.DMA,
            pltpu.SemaphoreType.DMA,
            pltpu.SemaphoreType.DMA,
            pltpu.SemaphoreType.DMA,
            pltpu.SemaphoreType.DMA,
        ],
    )
    def k(table_hbm, idx_hbm, out_hbm, idx_v,
          sl0, sl1, sl2, sl3, ob0, ob1, s0, s1, s2, s3, oa, ob):
        wid = lax.axis_index("s") * 2 + lax.axis_index("c")
        slots = (sl0, sl1, sl2, sl3)
        obufs = (ob0, ob1)
        gsems = (s0, s1, s2, s3)
        osems = (oa, ob)
        base_box = wid * BPT

        pltpu.sync_copy(idx_hbm.at[pl.ds(base_box, BPT)], idx_v)

        def issue(slot_i, box_local, half):
            pltpu.async_copy(
                table_hbm.at[idx_v.at[box_local, half]],
                slots[slot_i],
                gsems[slot_i],
            )

        for i in range(DEPTH):
            issue(i, i // 2, i % 2)

        def compute_chunk(slot_i, obuf_i, half):
            nbins = 26 if half == 0 else P * P - 26
            soff = 26 * half

            sbuf = slots[slot_i]
            obuf = obufs[obuf_i]

            def bin_body(sl, carry):
                rr = sl * 4
                for hh in range(CH // 16):
                    cs = pl.ds(hh * 16, 16)
                    g0 = plsc.bitcast(sbuf[rr, cs], jnp.bfloat16)
                    g1 = plsc.bitcast(sbuf[rr + 1, cs], jnp.bfloat16)
                    g2 = plsc.bitcast(sbuf[rr + 2, cs], jnp.bfloat16)
                    g3 = plsc.bitcast(sbuf[rr + 3, cs], jnp.bfloat16)
                    m = jnp.maximum(jnp.maximum(g0, g1), jnp.maximum(g2, g3))
                    obuf[soff + sl, cs] = plsc.bitcast(m, jnp.float32)
                return carry

            lax.fori_loop(0, nbins, bin_body, 0)

        nsteps = BPT // 2

        def step_body(g, carry):
            for i in range(DEPTH):
                obuf_i = i // 2
                pltpu.make_async_copy(
                    table_hbm.at[idx_v.at[0, 0]], slots[i], gsems[i]
                ).wait()
                if i % 2 == 0:
                    @pl.when(g > 0)
                    def _():
                        pltpu.make_async_copy(
                            obufs[obuf_i], out_hbm.at[0], osems[obuf_i]
                        ).wait()

                compute_chunk(i, obuf_i, i % 2)

                @pl.when(g < nsteps - 1)
                def _():
                    issue(i, (g + 1) * 2 + i // 2, i % 2)

                if i % 2 == 1:
                    box = base_box + g * 2 + obuf_i
                    pltpu.async_copy(
                        obufs[obuf_i], out_hbm.at[box], osems[obuf_i]
                    )
            return carry

        lax.fori_loop(0, nsteps, step_body, 0)
        for obuf_i in range(2):
            pltpu.make_async_copy(
                obufs[obuf_i], out_hbm.at[0], osems[obuf_i]
            ).wait()

    return k(table, idx)


def _transpose_out(gathered, n):
    blk = 8

    def body(in_ref, out_ref):
        xi = lax.bitcast_convert_type(in_ref[...][:, :, :CH], jnp.int32)
        lo = lax.bitcast_convert_type(xi << 16, jnp.float32)
        hi = lax.bitcast_convert_type(
            xi & jnp.int32(-65536), jnp.float32
        )
        x = jnp.concatenate([lo, hi], axis=2)
        out_ref[...] = jnp.transpose(x, (0, 2, 1))

    return pl.pallas_call(
        body,
        grid=(n // blk,),
        in_specs=[pl.BlockSpec((blk, P * P, CW), lambda i: (i, 0, 0))],
        out_specs=pl.BlockSpec((blk, C, P * P), lambda i: (i, 0, 0)),
        out_shape=jax.ShapeDtypeStruct((n, C, P * P), jnp.float32),
    )(gathered)


_COLS = None


def _slot_cols():
    global _COLS
    if _COLS is None:
        import numpy as np

        s = np.arange(256)
        b = s >> 2
        _COLS = np.stack([b // P, b % P, s & 3]).astype(np.int32)
    return _COLS


def kernel(feature, boxes, image_size):
    n = boxes.shape[0]
    ih = image_size[0].astype(jnp.float32)
    iw = image_size[1].astype(jnp.float32)
    scale = jnp.minimum(jnp.float32(H), jnp.float32(W)) / jnp.minimum(ih, iw)

    npad = NW * BPT
    rbox = jnp.round(boxes * scale).astype(jnp.int32)

    table, idx = _build_tables_and_idx(
        feature[0], rbox, jnp.asarray(_slot_cols()), n, npad
    )
    gathered = _sc_gather_max(table, idx.reshape(npad, 2, GCH), npad)
    out = _transpose_out(gathered[:n], n)
    return out.reshape(n, C, P, P)


# speedup vs baseline: 16.3867x; 1.2378x over previous
import functools

import jax
import jax.numpy as jnp
from jax import lax
from jax.experimental import pallas as pl
from jax.experimental.pallas import tpu as pltpu
from jax.experimental.pallas import tpu_sc as plsc

P = 7
H = 32
W = 32
C = 192
CH = 96
CW = 128
NLVL = 9
TROWS = NLVL * H * W
NZROWS = 256
ZROW = TROWS
TPAD = TROWS + NZROWS
RPB = 4 * P * P
RPB_PAD = 208
GCH = 104
NW = 32
BPT = 33
DEPTH = 6
NOB = DEPTH // 2


def _idx_math(rb, boxn, slot, ph, pw, jj, n):
    rsw = rb[:, 0:1]
    rsh = rb[:, 1:2]
    rew = rb[:, 2:3]
    reh = rb[:, 3:4]
    roi_w = jnp.maximum(rew - rsw + 1, 1)
    roi_h = jnp.maximum(reh - rsh + 1, 1)
    inv7 = jnp.float32(1.0 / 7.0)

    def fdiv7(x):
        return jnp.floor((x.astype(jnp.float32) + 0.5) * inv7).astype(jnp.int32)

    def seg(pb, rs, roi, lim):
        start = jnp.clip(fdiv7(pb * roi) + rs, 0, lim)
        end = jnp.clip(fdiv7((pb + 1) * roi - 1) + 1 + rs, 0, lim)
        ln = end - start
        pw2 = jnp.where(ln >= 4, 4, jnp.where(ln >= 2, 2, 1))
        k = jnp.where(ln >= 4, 2, jnp.where(ln >= 2, 1, 0))
        return start, end - pw2, k, ln <= 0

    ha, hb, kh, eh = seg(ph, rsh, roi_h, H)
    wa, wb, kw, ew = seg(pw, rsw, roi_w, W)

    base = (kh * 3 + kw) * (H * W)
    ih = jnp.where(jj >= 2, hb, ha) * W
    iw = jnp.where((jj & 1) == 1, wb, wa)
    idx = base + ih + iw
    zspread = ZROW + ((boxn * 83 + slot) & (NZROWS - 1))
    bad = eh | ew | (boxn >= n) | (slot >= RPB)
    return jnp.clip(jnp.where(bad, zspread, idx), 0, TPAD - 1)


def _pack(wk):
    lo = lax.bitcast_convert_type(wk[:, :CH].astype(jnp.bfloat16), jnp.uint16)
    hi = lax.bitcast_convert_type(wk[:, CH:C].astype(jnp.bfloat16), jnp.uint16)
    w = (hi.astype(jnp.uint32) << 16) | lo.astype(jnp.uint32)
    w = lax.bitcast_convert_type(w, jnp.float32)
    return jnp.concatenate(
        [w, jnp.zeros((wk.shape[0], CW - CH), jnp.float32)], axis=1
    )


def _build_tables_and_idx(fmap, rbox, cols, n, npad):
    nslot = 256

    def body(f_ref, b_ref, c_ref, out_ref, idx_ref):
        rb = jnp.concatenate(
            [b_ref[...], jnp.zeros((npad - n, 4), jnp.int32)], axis=0
        )
        cc = c_ref[...]
        ph = jnp.broadcast_to(cc[0:1, :], (npad, nslot))
        pw = jnp.broadcast_to(cc[1:2, :], (npad, nslot))
        jj = jnp.broadcast_to(cc[2:3, :], (npad, nslot))
        boxn = lax.broadcasted_iota(jnp.int32, (npad, nslot), 0)
        slot = lax.broadcasted_iota(jnp.int32, (npad, nslot), 1)
        idx = _idx_math(rb, boxn, slot, ph, pw, jj, n)
        idx_ref[...] = idx[:, :RPB_PAD]

        _tables_body(f_ref, out_ref)

    return pl.pallas_call(
        body,
        out_shape=(
            jax.ShapeDtypeStruct((TPAD, CW), jnp.float32),
            jax.ShapeDtypeStruct((npad, RPB_PAD), jnp.int32),
        ),
    )(fmap, rbox, cols)


def _tables_body(f_ref, out_ref):
    x = f_ref[...]
    xt = jnp.transpose(x.reshape(C, H * W))
    x3 = xt.reshape(H, W, C)

    def shift_h(a, d):
        tail = jnp.broadcast_to(a[H - 1 :], (d, W, C))
        return jnp.concatenate([a[d:], tail], axis=0)

    def shift_w(a, d):
        tail = jnp.broadcast_to(a[:, W - 1 :], (H, d, C))
        return jnp.concatenate([a[:, d:], tail], axis=1)

    h0 = x3
    h1 = jnp.maximum(h0, shift_h(h0, 1))
    h2 = jnp.maximum(h1, shift_h(h1, 2))
    for kh, hk in enumerate((h0, h1, h2)):
        w0 = hk
        w1 = jnp.maximum(w0, shift_w(w0, 1))
        w2 = jnp.maximum(w1, shift_w(w1, 2))
        for kw, wk in enumerate((w0, w1, w2)):
            lvl = kh * 3 + kw
            out_ref[pl.ds(lvl * H * W, H * W), :] = _pack(wk.reshape(H * W, C))
    out_ref[pl.ds(TROWS, NZROWS), :] = jnp.zeros((NZROWS, CW), jnp.float32)


def _sc_gather_max(table, idx, npad):
    mesh = plsc.VectorSubcoreMesh(core_axis_name="c", subcore_axis_name="s")

    @functools.partial(
        pl.kernel,
        mesh=mesh,
        compiler_params=pltpu.CompilerParams(needs_layout_passes=False),
        out_type=jax.ShapeDtypeStruct((npad, P * P, CW), jnp.float32),
        scratch_types=(
            [pltpu.VMEM((BPT, 2, GCH), jnp.int32)]
            + [pltpu.VMEM((GCH, CW), jnp.float32)] * DEPTH
            + [pltpu.VMEM((P * P, CW), jnp.float32)] * NOB
            + [pltpu.SemaphoreType.DMA] * (DEPTH + NOB)
        ),
    )
    def k(table_hbm, idx_hbm, out_hbm, idx_v, *bufs):
        wid = lax.axis_index("s") * 2 + lax.axis_index("c")
        slots = bufs[:DEPTH]
        obufs = bufs[DEPTH : DEPTH + NOB]
        gsems = bufs[DEPTH + NOB : 2 * DEPTH + NOB]
        osems = bufs[2 * DEPTH + NOB :]
        base_box = wid * BPT

        pltpu.sync_copy(idx_hbm.at[pl.ds(base_box, BPT)], idx_v)

        def issue(slot_i, box_local, half):
            pltpu.async_copy(
                table_hbm.at[idx_v.at[box_local, half]],
                slots[slot_i],
                gsems[slot_i],
            )

        for i in range(DEPTH):
            issue(i, i // 2, i % 2)

        def compute_chunk(slot_i, obuf_i, half):
            nbins = 26 if half == 0 else P * P - 26
            soff = 26 * half

            sbuf = slots[slot_i]
            obuf = obufs[obuf_i]

            def bin_body(sl, carry):
                rr = sl * 4
                for hh in range(CH // 16):
                    cs = pl.ds(hh * 16, 16)
                    g0 = plsc.bitcast(sbuf[rr, cs], jnp.bfloat16)
                    g1 = plsc.bitcast(sbuf[rr + 1, cs], jnp.bfloat16)
                    g2 = plsc.bitcast(sbuf[rr + 2, cs], jnp.bfloat16)
                    g3 = plsc.bitcast(sbuf[rr + 3, cs], jnp.bfloat16)
                    m = jnp.maximum(jnp.maximum(g0, g1), jnp.maximum(g2, g3))
                    obuf[soff + sl, cs] = plsc.bitcast(m, jnp.float32)
                return carry

            lax.fori_loop(0, nbins, bin_body, 0)

        nsteps = 2 * BPT // DEPTH

        def step_body(g, carry):
            for i in range(DEPTH):
                obuf_i = i // 2
                pltpu.make_async_copy(
                    table_hbm.at[idx_v.at[0, 0]], slots[i], gsems[i]
                ).wait()
                if i % 2 == 0:
                    @pl.when(g > 0)
                    def _():
                        pltpu.make_async_copy(
                            obufs[obuf_i], out_hbm.at[0], osems[obuf_i]
                        ).wait()

                compute_chunk(i, obuf_i, i % 2)

                @pl.when(g < nsteps - 1)
                def _():
                    issue(i, (g + 1) * NOB + i // 2, i % 2)

                if i % 2 == 1:
                    box = base_box + g * NOB + obuf_i
                    pltpu.async_copy(
                        obufs[obuf_i], out_hbm.at[box], osems[obuf_i]
                    )
            return carry

        lax.fori_loop(0, nsteps, step_body, 0)
        for obuf_i in range(NOB):
            pltpu.make_async_copy(
                obufs[obuf_i], out_hbm.at[0], osems[obuf_i]
            ).wait()

    return k(table, idx)


def _transpose_out(gathered, n):
    blk = 40

    def body(in_ref, out_ref):
        xi = lax.bitcast_convert_type(in_ref[...][:, :, :CH], jnp.int32)
        lo = lax.bitcast_convert_type(xi << 16, jnp.float32)
        hi = lax.bitcast_convert_type(
            xi & jnp.int32(-65536), jnp.float32
        )
        x = jnp.concatenate([lo, hi], axis=2)
        out_ref[...] = jnp.transpose(x, (0, 2, 1))

    return pl.pallas_call(
        body,
        grid=(n // blk,),
        in_specs=[pl.BlockSpec((blk, P * P, CW), lambda i: (i, 0, 0))],
        out_specs=pl.BlockSpec((blk, C, P * P), lambda i: (i, 0, 0)),
        out_shape=jax.ShapeDtypeStruct((n, C, P * P), jnp.float32),
    )(gathered)


_COLS = None


def _slot_cols():
    global _COLS
    if _COLS is None:
        import numpy as np

        s = np.arange(256)
        b = s >> 2
        _COLS = np.stack([b // P, b % P, s & 3]).astype(np.int32)
    return _COLS


def kernel(feature, boxes, image_size):
    n = boxes.shape[0]
    ih = image_size[0].astype(jnp.float32)
    iw = image_size[1].astype(jnp.float32)
    scale = jnp.minimum(jnp.float32(H), jnp.float32(W)) / jnp.minimum(ih, iw)

    npad = NW * BPT
    rbox = jnp.round(boxes * scale).astype(jnp.int32)

    table, idx = _build_tables_and_idx(
        feature[0], rbox, jnp.asarray(_slot_cols()), n, npad
    )
    gathered = _sc_gather_max(table, idx.reshape(npad, 2, GCH), npad)
    out = _transpose_out(gathered[:n], n)
    return out.reshape(n, C, P, P)


# speedup vs baseline: 17.3360x; 1.0579x over previous
import functools

import jax
import jax.numpy as jnp
from jax import lax
from jax.experimental import pallas as pl
from jax.experimental.pallas import tpu as pltpu
from jax.experimental.pallas import tpu_sc as plsc

P = 7
H = 32
W = 32
C = 192
CH = 96
CW = 128
NLVL = 9
TROWS = NLVL * H * W
NZROWS = 256
ZROW = TROWS
TPAD = TROWS + NZROWS
RPB = 4 * P * P
RPB_PAD = 208
GCH = 104
NW = 32
BPT = 33
DEPTH = 6
NOB = DEPTH // 2


def _idx_math(rb, boxn, slot, ph, pw, jj, n):
    rsw = rb[:, 0:1]
    rsh = rb[:, 1:2]
    rew = rb[:, 2:3]
    reh = rb[:, 3:4]
    roi_w = jnp.maximum(rew - rsw + 1, 1)
    roi_h = jnp.maximum(reh - rsh + 1, 1)
    inv7 = jnp.float32(1.0 / 7.0)

    def fdiv7(x):
        return jnp.floor((x.astype(jnp.float32) + 0.5) * inv7).astype(jnp.int32)

    def seg(pb, rs, roi, lim):
        start = jnp.clip(fdiv7(pb * roi) + rs, 0, lim)
        end = jnp.clip(fdiv7((pb + 1) * roi - 1) + 1 + rs, 0, lim)
        ln = end - start
        pw2 = jnp.where(ln >= 4, 4, jnp.where(ln >= 2, 2, 1))
        k = jnp.where(ln >= 4, 2, jnp.where(ln >= 2, 1, 0))
        return start, end - pw2, k, ln <= 0

    ha, hb, kh, eh = seg(ph, rsh, roi_h, H)
    wa, wb, kw, ew = seg(pw, rsw, roi_w, W)

    base = (kh * 3 + kw) * (H * W)
    ih = jnp.where(jj >= 2, hb, ha) * W
    iw = jnp.where((jj & 1) == 1, wb, wa)
    idx = base + ih + iw
    zspread = ZROW + ((boxn * 83 + slot) & (NZROWS - 1))
    bad = eh | ew | (boxn >= n) | (slot >= RPB)
    return jnp.clip(jnp.where(bad, zspread, idx), 0, TPAD - 1)


def _pack(wk):
    lo = lax.bitcast_convert_type(wk[:, :CH].astype(jnp.bfloat16), jnp.uint16)
    hi = lax.bitcast_convert_type(wk[:, CH:C].astype(jnp.bfloat16), jnp.uint16)
    w = (hi.astype(jnp.uint32) << 16) | lo.astype(jnp.uint32)
    w = lax.bitcast_convert_type(w, jnp.float32)
    return jnp.concatenate(
        [w, jnp.zeros((wk.shape[0], CW - CH), jnp.float32)], axis=1
    )


def _build_tables_and_idx(fmap, rbox, cols, n, npad):
    nslot = 256

    def body(f_ref, b_ref, c_ref, out_ref, idx_ref):
        rb = jnp.concatenate(
            [b_ref[...], jnp.zeros((npad - n, 4), jnp.int32)], axis=0
        )
        cc = c_ref[...]
        ph = jnp.broadcast_to(cc[0:1, :], (npad, nslot))
        pw = jnp.broadcast_to(cc[1:2, :], (npad, nslot))
        jj = jnp.broadcast_to(cc[2:3, :], (npad, nslot))
        boxn = lax.broadcasted_iota(jnp.int32, (npad, nslot), 0)
        slot = lax.broadcasted_iota(jnp.int32, (npad, nslot), 1)
        idx = _idx_math(rb, boxn, slot, ph, pw, jj, n)
        idx_ref[...] = idx[:, :RPB_PAD]

        _tables_body(f_ref, out_ref)

    return pl.pallas_call(
        body,
        out_shape=(
            jax.ShapeDtypeStruct((TPAD, CW), jnp.float32),
            jax.ShapeDtypeStruct((npad, RPB_PAD), jnp.int32),
        ),
    )(fmap, rbox, cols)


def _tables_body(f_ref, out_ref):
    x = f_ref[...]
    xt = jnp.transpose(x.reshape(C, H * W))
    x3 = xt.reshape(H, W, C)

    def shift_h(a, d):
        tail = jnp.broadcast_to(a[H - 1 :], (d, W, C))
        return jnp.concatenate([a[d:], tail], axis=0)

    def shift_w(a, d):
        tail = jnp.broadcast_to(a[:, W - 1 :], (H, d, C))
        return jnp.concatenate([a[:, d:], tail], axis=1)

    h0 = x3
    h1 = jnp.maximum(h0, shift_h(h0, 1))
    h2 = jnp.maximum(h1, shift_h(h1, 2))
    for kh, hk in enumerate((h0, h1, h2)):
        w0 = hk
        w1 = jnp.maximum(w0, shift_w(w0, 1))
        w2 = jnp.maximum(w1, shift_w(w1, 2))
        for kw, wk in enumerate((w0, w1, w2)):
            lvl = kh * 3 + kw
            out_ref[pl.ds(lvl * H * W, H * W), :] = _pack(wk.reshape(H * W, C))
    out_ref[pl.ds(TROWS, NZROWS), :] = jnp.zeros((NZROWS, CW), jnp.float32)


def _sc_gather_max(table, idx, npad):
    mesh = plsc.VectorSubcoreMesh(core_axis_name="c", subcore_axis_name="s")

    @functools.partial(
        pl.kernel,
        mesh=mesh,
        compiler_params=pltpu.CompilerParams(needs_layout_passes=False),
        out_type=jax.ShapeDtypeStruct((npad, P * P, CW), jnp.float32),
        scratch_types=(
            [pltpu.VMEM((BPT, 2, GCH), jnp.int32)]
            + [pltpu.VMEM((GCH, CW), jnp.float32)] * DEPTH
            + [pltpu.VMEM((P * P, CW), jnp.float32)] * NOB
            + [pltpu.SemaphoreType.DMA] * (DEPTH + NOB)
        ),
    )
    def k(table_hbm, idx_hbm, out_hbm, idx_v, *bufs):
        wid = lax.axis_index("s") * 2 + lax.axis_index("c")
        slots = bufs[:DEPTH]
        obufs = bufs[DEPTH : DEPTH + NOB]
        gsems = bufs[DEPTH + NOB : 2 * DEPTH + NOB]
        osems = bufs[2 * DEPTH + NOB :]
        base_box = wid * BPT

        pltpu.sync_copy(idx_hbm.at[pl.ds(base_box, BPT)], idx_v)

        def issue(slot_i, box_local, half):
            pltpu.async_copy(
                table_hbm.at[idx_v.at[box_local, half]],
                slots[slot_i],
                gsems[slot_i],
            )

        for i in range(DEPTH):
            issue(i, i // 2, i % 2)

        def compute_chunk(slot_i, obuf_i, half):
            nbins = 26 if half == 0 else P * P - 26
            soff = 26 * half

            sbuf = slots[slot_i]
            obuf = obufs[obuf_i]

            def bin_body(sl, carry):
                rr = sl * 4
                for hh in range(CH // 16):
                    cs = pl.ds(hh * 16, 16)
                    g0 = plsc.bitcast(sbuf[rr, cs], jnp.bfloat16)
                    g1 = plsc.bitcast(sbuf[rr + 1, cs], jnp.bfloat16)
                    g2 = plsc.bitcast(sbuf[rr + 2, cs], jnp.bfloat16)
                    g3 = plsc.bitcast(sbuf[rr + 3, cs], jnp.bfloat16)
                    m = jnp.maximum(jnp.maximum(g0, g1), jnp.maximum(g2, g3))
                    obuf[soff + sl, cs] = plsc.bitcast(m, jnp.float32)
                return carry

            lax.fori_loop(0, nbins, bin_body, 0)

        nsteps = 2 * BPT // DEPTH

        def step_body(g, carry):
            for i in range(DEPTH):
                obuf_i = i // 2
                pltpu.make_async_copy(
                    table_hbm.at[idx_v.at[0, 0]], slots[i], gsems[i]
                ).wait()
                if i % 2 == 0:
                    @pl.when(g > 0)
                    def _():
                        pltpu.make_async_copy(
                            obufs[obuf_i], out_hbm.at[0], osems[obuf_i]
                        ).wait()

                compute_chunk(i, obuf_i, i % 2)

                @pl.when(g < nsteps - 1)
                def _():
                    issue(i, (g + 1) * NOB + i // 2, i % 2)

                if i % 2 == 1:
                    box = base_box + g * NOB + obuf_i
                    pltpu.async_copy(
                        obufs[obuf_i], out_hbm.at[box], osems[obuf_i]
                    )
            return carry

        lax.fori_loop(0, nsteps, step_body, 0)
        for obuf_i in range(NOB):
            pltpu.make_async_copy(
                obufs[obuf_i], out_hbm.at[0], osems[obuf_i]
            ).wait()

    return k(table, idx)


def _transpose_out(gathered, n):
    blk = 40

    def body(in_ref, out_ref):
        xi = lax.bitcast_convert_type(in_ref[...][:, :, :CH], jnp.int32)
        lo = lax.bitcast_convert_type(xi << 16, jnp.float32)
        hi = lax.bitcast_convert_type(
            xi & jnp.int32(-65536), jnp.float32
        )
        x = jnp.concatenate([lo, hi], axis=2)
        out_ref[...] = jnp.transpose(x, (0, 2, 1))

    return pl.pallas_call(
        body,
        grid=(n // blk,),
        in_specs=[pl.BlockSpec((blk, P * P, CW), lambda i: (i, 0, 0))],
        out_specs=pl.BlockSpec((blk, C, P * P), lambda i: (i, 0, 0)),
        out_shape=jax.ShapeDtypeStruct((n, C, P * P), jnp.float32),
    )(gathered)


_COLS = None


def _slot_cols():
    global _COLS
    if _COLS is None:
        import numpy as np

        s = np.arange(256)
        b = s >> 2
        _COLS = np.stack([b // P, b % P, s & 3]).astype(np.int32)
    return _COLS


def kernel(feature, boxes, image_size):
    n = boxes.shape[0]
    ih = image_size[0].astype(jnp.float32)
    iw = image_size[1].astype(jnp.float32)
    scale = jnp.minimum(jnp.float32(H), jnp.float32(W)) / jnp.minimum(ih, iw)

    npad = NW * BPT
    rbox = jnp.round(boxes * scale).astype(jnp.int32)

    table, idx = _build_tables_and_idx(
        feature[0], rbox, jnp.asarray(_slot_cols()), n, npad
    )
    gathered = _sc_gather_max(table, idx.reshape(npad, 2, GCH), npad)
    return _transpose_out(gathered, n).reshape(n, C, P, P)


# speedup vs baseline: 23.4433x; 1.3523x over previous
import functools

import jax
import jax.numpy as jnp
from jax import lax
from jax.experimental import pallas as pl
from jax.experimental.pallas import tpu as pltpu
from jax.experimental.pallas import tpu_sc as plsc

P = 7
H = 32
W = 32
C = 192
CH = 96
CW = 128
NLVL = 9
TROWS = NLVL * H * W
NZROWS = 256
ZROW = TROWS
TPAD = TROWS + NZROWS
RPB = 4 * P * P
RPB_PAD = 208
GCH = 104
NW = 32
BPT = 33
DEPTH = 6
NOB = DEPTH // 2


def _idx_math(rb, boxn, slot, ph, pw, jj, n):
    rsw = rb[:, 0:1]
    rsh = rb[:, 1:2]
    rew = rb[:, 2:3]
    reh = rb[:, 3:4]
    roi_w = jnp.maximum(rew - rsw + 1, 1)
    roi_h = jnp.maximum(reh - rsh + 1, 1)
    inv7 = jnp.float32(1.0 / 7.0)

    def fdiv7(x):
        return jnp.floor((x.astype(jnp.float32) + 0.5) * inv7).astype(jnp.int32)

    def seg(pb, rs, roi, lim):
        start = jnp.clip(fdiv7(pb * roi) + rs, 0, lim)
        end = jnp.clip(fdiv7((pb + 1) * roi - 1) + 1 + rs, 0, lim)
        ln = end - start
        pw2 = jnp.where(ln >= 4, 4, jnp.where(ln >= 2, 2, 1))
        k = jnp.where(ln >= 4, 2, jnp.where(ln >= 2, 1, 0))
        return start, end - pw2, k, ln <= 0

    ha, hb, kh, eh = seg(ph, rsh, roi_h, H)
    wa, wb, kw, ew = seg(pw, rsw, roi_w, W)

    base = (kh * 3 + kw) * (H * W)
    ih = jnp.where(jj >= 2, hb, ha) * W
    iw = jnp.where((jj & 1) == 1, wb, wa)
    idx = base + ih + iw
    zspread = ZROW + ((boxn * 83 + slot) & (NZROWS - 1))
    bad = eh | ew | (boxn >= n) | (slot >= RPB)
    return jnp.clip(jnp.where(bad, zspread, idx), 0, TPAD - 1)


def _pack(wk):
    lo = lax.bitcast_convert_type(wk[:, :CH].astype(jnp.bfloat16), jnp.uint16)
    hi = lax.bitcast_convert_type(wk[:, CH:C].astype(jnp.bfloat16), jnp.uint16)
    w = (hi.astype(jnp.uint32) << 16) | lo.astype(jnp.uint32)
    w = lax.bitcast_convert_type(w, jnp.float32)
    return jnp.concatenate(
        [w, jnp.zeros((wk.shape[0], CW - CH), jnp.float32)], axis=1
    )


def _build_tables_and_idx(fmap, rbox, cols, n, npad):
    nslot = 256

    def body(f_ref, b_ref, c_ref, out_ref, idx_ref):
        rb = jnp.concatenate(
            [b_ref[...], jnp.zeros((npad - n, 4), jnp.int32)], axis=0
        )
        cc = c_ref[...]
        ph = jnp.broadcast_to(cc[0:1, :], (npad, nslot))
        pw = jnp.broadcast_to(cc[1:2, :], (npad, nslot))
        jj = jnp.broadcast_to(cc[2:3, :], (npad, nslot))
        boxn = lax.broadcasted_iota(jnp.int32, (npad, nslot), 0)
        slot = lax.broadcasted_iota(jnp.int32, (npad, nslot), 1)
        idx = _idx_math(rb, boxn, slot, ph, pw, jj, n)
        idx_ref[...] = idx[:, :RPB_PAD]

        _tables_body(f_ref, out_ref)

    return pl.pallas_call(
        body,
        out_shape=(
            jax.ShapeDtypeStruct((TPAD, CW), jnp.float32),
            jax.ShapeDtypeStruct((npad, RPB_PAD), jnp.int32),
        ),
    )(fmap, rbox, cols)


def _tables_body(f_ref, out_ref):
    x = f_ref[...]
    xt = jnp.transpose(x.reshape(C, H * W))
    x3 = xt.reshape(H, W, C)

    def shift_h(a, d):
        tail = jnp.broadcast_to(a[H - 1 :], (d, W, C))
        return jnp.concatenate([a[d:], tail], axis=0)

    def shift_w(a, d):
        tail = jnp.broadcast_to(a[:, W - 1 :], (H, d, C))
        return jnp.concatenate([a[:, d:], tail], axis=1)

    h0 = x3
    h1 = jnp.maximum(h0, shift_h(h0, 1))
    h2 = jnp.maximum(h1, shift_h(h1, 2))
    for kh, hk in enumerate((h0, h1, h2)):
        w0 = hk
        w1 = jnp.maximum(w0, shift_w(w0, 1))
        w2 = jnp.maximum(w1, shift_w(w1, 2))
        for kw, wk in enumerate((w0, w1, w2)):
            lvl = kh * 3 + kw
            out_ref[pl.ds(lvl * H * W, H * W), :] = _pack(wk.reshape(H * W, C))
    out_ref[pl.ds(TROWS, NZROWS), :] = jnp.zeros((NZROWS, CW), jnp.float32)


def _sc_gather_max(table, idx, npad):
    mesh = plsc.VectorSubcoreMesh(core_axis_name="c", subcore_axis_name="s")

    @functools.partial(
        pl.kernel,
        mesh=mesh,
        compiler_params=pltpu.CompilerParams(needs_layout_passes=False),
        out_type=jax.ShapeDtypeStruct((npad, P * P, CW), jnp.float32),
        scratch_types=(
            [pltpu.VMEM((BPT, 2, GCH), jnp.int32)]
            + [pltpu.VMEM((GCH, CW), jnp.float32)] * DEPTH
            + [pltpu.VMEM((P * P, CW), jnp.float32)] * NOB
            + [pltpu.SemaphoreType.DMA] * (DEPTH + NOB)
        ),
    )
    def k(table_hbm, idx_hbm, out_hbm, idx_v, *bufs):
        wid = lax.axis_index("s") * 2 + lax.axis_index("c")
        slots = bufs[:DEPTH]
        obufs = bufs[DEPTH : DEPTH + NOB]
        gsems = bufs[DEPTH + NOB : 2 * DEPTH + NOB]
        osems = bufs[2 * DEPTH + NOB :]
        base_box = wid * BPT

        pltpu.sync_copy(idx_hbm.at[pl.ds(base_box, BPT)], idx_v)

        def issue(slot_i, box_local, half):
            pltpu.async_copy(
                table_hbm.at[idx_v.at[box_local, half]],
                slots[slot_i],
                gsems[slot_i],
            )

        for i in range(DEPTH):
            issue(i, i // 2, i % 2)

        def compute_chunk(slot_i, obuf_i, half):
            nbins = 26 if half == 0 else P * P - 26
            soff = 26 * half

            sbuf = slots[slot_i]
            obuf = obufs[obuf_i]

            def bin_body(sl, carry):
                rr = sl * 4
                for hh in range(CH // 16):
                    cs = pl.ds(hh * 16, 16)
                    g0 = plsc.bitcast(sbuf[rr, cs], jnp.bfloat16)
                    g1 = plsc.bitcast(sbuf[rr + 1, cs], jnp.bfloat16)
                    g2 = plsc.bitcast(sbuf[rr + 2, cs], jnp.bfloat16)
                    g3 = plsc.bitcast(sbuf[rr + 3, cs], jnp.bfloat16)
                    m = jnp.maximum(jnp.maximum(g0, g1), jnp.maximum(g2, g3))
                    obuf[soff + sl, cs] = plsc.bitcast(m, jnp.float32)
                return carry

            lax.fori_loop(0, nbins, bin_body, 0)

        nsteps = 2 * BPT // DEPTH

        def step_body(g, carry):
            for i in range(DEPTH):
                obuf_i = i // 2
                pltpu.make_async_copy(
                    table_hbm.at[idx_v.at[0, 0]], slots[i], gsems[i]
                ).wait()
                if i % 2 == 0:
                    @pl.when(g > 0)
                    def _():
                        pltpu.make_async_copy(
                            obufs[obuf_i], out_hbm.at[0], osems[obuf_i]
                        ).wait()

                compute_chunk(i, obuf_i, i % 2)

                @pl.when(g < nsteps - 1)
                def _():
                    issue(i, (g + 1) * NOB + i // 2, i % 2)

                if i % 2 == 1:
                    box = base_box + g * NOB + obuf_i
                    pltpu.async_copy(
                        obufs[obuf_i], out_hbm.at[box], osems[obuf_i]
                    )
            return carry

        lax.fori_loop(0, nsteps, step_body, 0)
        for obuf_i in range(NOB):
            pltpu.make_async_copy(
                obufs[obuf_i], out_hbm.at[0], osems[obuf_i]
            ).wait()

    return k(table, idx)


def _transpose_out(gathered, n):
    blk = 128

    def body(in_ref, out_ref):
        for s in range(P * P):
            xs = jnp.transpose(in_ref[:, s, :])
            xi = lax.bitcast_convert_type(xs, jnp.int32)
            lo = lax.bitcast_convert_type(xi << 16, jnp.float32)
            hi = lax.bitcast_convert_type(xi & jnp.int32(-65536), jnp.float32)
            out_ref[s, 0:CH, :] = lo[0:CH, :]
            out_ref[s, CH:C, :] = hi[0:CH, :]

    nblk = -(-n // blk)
    return pl.pallas_call(
        body,
        grid=(nblk,),
        in_specs=[pl.BlockSpec((blk, P * P, CW), lambda i: (i, 0, 0))],
        out_specs=pl.BlockSpec((P * P, C, blk), lambda i: (0, 0, i)),
        out_shape=jax.ShapeDtypeStruct((P * P, C, n), jnp.float32),
    )(gathered)


_COLS = None


def _slot_cols():
    global _COLS
    if _COLS is None:
        import numpy as np

        s = np.arange(256)
        b = s >> 2
        _COLS = np.stack([b // P, b % P, s & 3]).astype(np.int32)
    return _COLS


def kernel(feature, boxes, image_size):
    n = boxes.shape[0]
    ih = image_size[0].astype(jnp.float32)
    iw = image_size[1].astype(jnp.float32)
    scale = jnp.minimum(jnp.float32(H), jnp.float32(W)) / jnp.minimum(ih, iw)

    npad = NW * BPT
    rbox = jnp.round(boxes * scale).astype(jnp.int32)

    table, idx = _build_tables_and_idx(
        feature[0], rbox, jnp.asarray(_slot_cols()), n, npad
    )
    gathered = _sc_gather_max(table, idx.reshape(npad, 2, GCH), npad)
    out = _transpose_out(gathered, n)
    return jnp.transpose(out, (2, 1, 0)).reshape(n, C, P, P)
